# Initial kernel scaffold; baseline (speedup 1.0000x reference)
import jax
import jax.numpy as jnp
from jax.experimental import pallas as pl


def kernel(feats, edge_index, W_self1, W_neigh1, b1, W_self2, W_neigh2, b2):
    raise NotImplementedError("write your pallas kernel here")


# speedup vs baseline: 4.2758x; 4.2758x over previous
import functools

import jax
import jax.numpy as jnp
from jax import lax
from jax.experimental import pallas as pl
from jax.experimental.pallas import tpu as pltpu
from jax.experimental.pallas import tpu_sc as plsc

N = 10000
NP = 10240
E = 320000
IN_DIM = 128
H_DIM = 256
OUT_DIM = 256

NC = 2
NS = 16
D1 = IN_DIM + 16
D2 = H_DIM // 2

C = 80
RPT = NP // NS


def _sc_segsum_l1(x_aug, src, dst, zeros_init):
    ept = E // (NC * NS)
    chunks = ept // C
    mesh = plsc.VectorSubcoreMesh(core_axis_name="c", subcore_axis_name="s")

    @functools.partial(
        pl.kernel,
        out_type=jax.ShapeDtypeStruct((NC, NP, D1), jnp.float32),
        mesh=mesh,
        compiler_params=pltpu.CompilerParams(use_tc_tiling_on_sc=False),
        scratch_types=[
            pltpu.VMEM((C,), jnp.int32),
            pltpu.VMEM((C,), jnp.int32),
            pltpu.VMEM((C, D1), jnp.float32),
            pltpu.VMEM_SHARED((NP, D1), jnp.float32),
            pltpu.SemaphoreType.DMA,
        ],
    )
    def k(x_hbm, src_hbm, dst_hbm, z_hbm, out_hbm, src_v, dst_v, rows_v,
          acc_sh, sem):
        c = lax.axis_index("c")
        s = lax.axis_index("s")
        wid = c * NS + s
        pltpu.sync_copy(z_hbm.at[pl.ds(s * RPT, RPT)],
                        acc_sh.at[pl.ds(s * RPT, RPT)])
        plsc.subcore_barrier()
        ebase = wid * ept

        def body(j, carry):
            off = ebase + j * C
            pltpu.sync_copy(src_hbm.at[pl.ds(off, C)], src_v)
            pltpu.sync_copy(dst_hbm.at[pl.ds(off, C)], dst_v)
            pltpu.async_copy(x_hbm.at[src_v], rows_v, sem).wait()
            pltpu.sync_copy(rows_v, acc_sh.at[dst_v], add=True)
            return carry

        lax.fori_loop(0, chunks, body, 0)
        plsc.subcore_barrier()
        pltpu.sync_copy(acc_sh.at[pl.ds(s * RPT, RPT)],
                        out_hbm.at[c, pl.ds(s * RPT, RPT)])

    return k(x_aug, src, dst, zeros_init)


def _sc_segsum_l2(x_halves, src, dst, zeros_init):
    ept = E // NS
    chunks = ept // C
    mesh = plsc.VectorSubcoreMesh(core_axis_name="c", subcore_axis_name="s")

    @functools.partial(
        pl.kernel,
        out_type=jax.ShapeDtypeStruct((NC, NP, D2), jnp.float32),
        mesh=mesh,
        compiler_params=pltpu.CompilerParams(use_tc_tiling_on_sc=False),
        scratch_types=[
            pltpu.VMEM((C,), jnp.int32),
            pltpu.VMEM((C,), jnp.int32),
            pltpu.VMEM((C, D2), jnp.float32),
            pltpu.VMEM_SHARED((NP, D2), jnp.float32),
            pltpu.SemaphoreType.DMA,
        ],
    )
    def k(x_hbm, src_hbm, dst_hbm, z_hbm, out_hbm, src_v, dst_v, rows_v,
          acc_sh, sem):
        c = lax.axis_index("c")
        s = lax.axis_index("s")
        pltpu.sync_copy(z_hbm.at[pl.ds(s * RPT, RPT)],
                        acc_sh.at[pl.ds(s * RPT, RPT)])
        plsc.subcore_barrier()
        ebase = s * ept

        def body(j, carry):
            off = ebase + j * C
            pltpu.sync_copy(src_hbm.at[pl.ds(off, C)], src_v)
            pltpu.sync_copy(dst_hbm.at[pl.ds(off, C)], dst_v)
            pltpu.async_copy(x_hbm.at[c].at[src_v], rows_v, sem).wait()
            pltpu.sync_copy(rows_v, acc_sh.at[dst_v], add=True)
            return carry

        lax.fori_loop(0, chunks, body, 0)
        plsc.subcore_barrier()
        pltpu.sync_copy(acc_sh.at[pl.ds(s * RPT, RPT)],
                        out_hbm.at[c, pl.ds(s * RPT, RPT)])

    return k(x_halves, src, dst, zeros_init)


R = 1000


def _tc1_body(x_ref, p_ref, ws_ref, wn_ref, b_ref, h_ref, inv_ref):
    acc = p_ref[0] + p_ref[1]
    deg = acc[:, IN_DIM:IN_DIM + 1]
    inv = 1.0 / jnp.maximum(deg, 1.0)
    agg = acc[:, :IN_DIM] * inv
    h = jnp.dot(x_ref[...], ws_ref[...], preferred_element_type=jnp.float32)
    h = h + jnp.dot(agg, wn_ref[...], preferred_element_type=jnp.float32)
    h = jnp.maximum(h + b_ref[...], 0.0)
    h_ref[0] = h[:, :D2]
    h_ref[1] = h[:, D2:]
    inv_ref[...] = jnp.broadcast_to(inv, (R, D2))


def _tc_layer1(x, part, Ws, Wn, b):
    return pl.pallas_call(
        _tc1_body,
        grid=(N // R,),
        in_specs=[
            pl.BlockSpec((R, IN_DIM), lambda i: (i, 0)),
            pl.BlockSpec((NC, R, D1), lambda i: (0, i, 0)),
            pl.BlockSpec((IN_DIM, H_DIM), lambda i: (0, 0)),
            pl.BlockSpec((IN_DIM, H_DIM), lambda i: (0, 0)),
            pl.BlockSpec((1, H_DIM), lambda i: (0, 0)),
        ],
        out_specs=[
            pl.BlockSpec((NC, R, D2), lambda i: (0, i, 0)),
            pl.BlockSpec((R, D2), lambda i: (i, 0)),
        ],
        out_shape=[
            jax.ShapeDtypeStruct((NC, N, D2), jnp.float32),
            jax.ShapeDtypeStruct((N, D2), jnp.float32),
        ],
    )(x, part, Ws, Wn, b)


def _tc2_body(h_ref, q_ref, inv_ref, ws_ref, wn_ref, b_ref, o_ref):
    inv = inv_ref[...]
    out = jnp.dot(h_ref[0], ws_ref[:D2], preferred_element_type=jnp.float32)
    out = out + jnp.dot(h_ref[1], ws_ref[D2:],
                        preferred_element_type=jnp.float32)
    out = out + jnp.dot(q_ref[0] * inv, wn_ref[:D2],
                        preferred_element_type=jnp.float32)
    out = out + jnp.dot(q_ref[1] * inv, wn_ref[D2:],
                        preferred_element_type=jnp.float32)
    o_ref[...] = out + b_ref[...]


def _tc_layer2(h, q, inv, Ws, Wn, b):
    return pl.pallas_call(
        _tc2_body,
        grid=(N // R,),
        in_specs=[
            pl.BlockSpec((NC, R, D2), lambda i: (0, i, 0)),
            pl.BlockSpec((NC, R, D2), lambda i: (0, i, 0)),
            pl.BlockSpec((R, D2), lambda i: (i, 0)),
            pl.BlockSpec((H_DIM, OUT_DIM), lambda i: (0, 0)),
            pl.BlockSpec((H_DIM, OUT_DIM), lambda i: (0, 0)),
            pl.BlockSpec((1, OUT_DIM), lambda i: (0, 0)),
        ],
        out_specs=pl.BlockSpec((R, OUT_DIM), lambda i: (i, 0)),
        out_shape=jax.ShapeDtypeStruct((N, OUT_DIM), jnp.float32),
    )(h, q, inv, Ws, Wn, b)


def kernel(feats, edge_index, W_self1, W_neigh1, b1, W_self2, W_neigh2, b2):
    src = edge_index[0]
    dst = edge_index[1]
    pad = jnp.concatenate(
        [jnp.ones((N, 1), jnp.float32), jnp.zeros((N, 15), jnp.float32)],
        axis=1)
    x_aug = jnp.concatenate([feats, pad], axis=1)
    part1 = _sc_segsum_l1(x_aug, src, dst, jnp.zeros((NP, D1), jnp.float32))
    h, inv = _tc_layer1(feats, part1, W_self1, W_neigh1, b1.reshape(1, -1))
    part2 = _sc_segsum_l2(h, src, dst, jnp.zeros((NP, D2), jnp.float32))
    return _tc_layer2(h, part2, inv, W_self2, W_neigh2, b2.reshape(1, -1))


# speedup vs baseline: 9.4663x; 2.2139x over previous
import functools

import jax
import jax.numpy as jnp
from jax import lax
from jax.experimental import pallas as pl
from jax.experimental.pallas import tpu as pltpu
from jax.experimental.pallas import tpu_sc as plsc

N = 10000
NP = 10240
E = 320000
IN_DIM = 128
H_DIM = 256
OUT_DIM = 256

NC = 2
NS = 16
D1 = IN_DIM + 16
D2 = H_DIM // 2

C = 40
NBUF = 5
IB = 10
RPT = NP // NS


def _sc_segsum(x, idx_all, zeros_init, D, split_edges_by_core):
    ept = E // (NC * NS) if split_edges_by_core else E // NS
    chunks = ept // C
    outer = chunks // IB
    mesh = plsc.VectorSubcoreMesh(core_axis_name="c", subcore_axis_name="s")

    @functools.partial(
        pl.kernel,
        out_type=jax.ShapeDtypeStruct((NC, NP, D), jnp.float32),
        mesh=mesh,
        compiler_params=pltpu.CompilerParams(use_tc_tiling_on_sc=False),
        scratch_types=[
            pltpu.VMEM((IB, 2, C), jnp.int32),
            pltpu.VMEM((NBUF, C, D), jnp.float32),
            pltpu.VMEM_SHARED((NP, D), jnp.float32),
            pltpu.SemaphoreType.DMA((IB,)),
            pltpu.SemaphoreType.DMA((NBUF,)),
            pltpu.SemaphoreType.DMA((NBUF,)),
        ],
    )
    def k(x_hbm, idx_hbm, z_hbm, out_hbm, idxv, rowsv, acc_sh,
          isem, gsem, ssem):
        c = lax.axis_index("c")
        s = lax.axis_index("s")
        wid = c * NS + s
        ebase = (wid if split_edges_by_core else s) * ept
        cbase = ebase // C
        table = x_hbm if split_edges_by_core else x_hbm.at[c]
        pltpu.sync_copy(z_hbm.at[pl.ds(s * RPT, RPT)],
                        acc_sh.at[pl.ds(s * RPT, RPT)])
        plsc.subcore_barrier()

        def idx_start(j, bi):
            pltpu.async_copy(idx_hbm.at[cbase + j], idxv.at[bi],
                             isem.at[bi])

        def idx_wait(j, bi):
            pltpu.make_async_copy(idx_hbm.at[cbase + j], idxv.at[bi],
                                  isem.at[bi]).wait()

        def gather_start(j, b, bi):
            pltpu.async_copy(table.at[idxv.at[bi, 0]], rowsv.at[b],
                             gsem.at[b])

        def gather_wait(j, b, bi):
            pltpu.make_async_copy(table.at[idxv.at[bi, 0]], rowsv.at[b],
                                  gsem.at[b]).wait()

        def scatter_start(j, b, bi):
            pltpu.async_copy(rowsv.at[b], acc_sh.at[idxv.at[bi, 1]],
                             ssem.at[b], add=True)

        def scatter_wait(j, b, bi):
            pltpu.make_async_copy(rowsv.at[b], acc_sh.at[idxv.at[bi, 1]],
                                  ssem.at[b]).wait()

        idx_start(0, 0)
        idx_start(1, 1)
        for m in range(IB):
            idx_start(m + 2, (m + 2) % IB)
            if m >= NBUF:
                scatter_wait(m - NBUF, m % NBUF, (m - NBUF) % IB)
            idx_wait(m, m)
            gather_start(m, m % NBUF, m)
            if m >= 2:
                gather_wait(m - 2, (m - 2) % NBUF, m - 2)
                scatter_start(m - 2, (m - 2) % NBUF, m - 2)

        def step(j, mm):
            idx_start(j + 2, (mm + 2) % IB)
            scatter_wait(j - NBUF, mm % NBUF, (mm - NBUF) % IB)
            idx_wait(j, mm)
            gather_start(j, mm % NBUF, mm)
            b2 = (mm - 2) % NBUF
            bi2 = (mm - 2) % IB
            gather_wait(j - 2, b2, bi2)
            scatter_start(j - 2, b2, bi2)

        def body(g, carry):
            for mm in range(IB):
                step(g * IB + mm, mm)
            return carry

        lax.fori_loop(1, outer, body, 0)

        for j in (chunks - 2, chunks - 1):
            b = j % NBUF
            bi = j % IB
            gather_wait(j, b, bi)
            scatter_start(j, b, bi)
        for b in range(NBUF):
            scatter_wait(0, b, 0)
        idx_wait(chunks, chunks % IB)
        idx_wait(chunks + 1, (chunks + 1) % IB)
        plsc.subcore_barrier()
        pltpu.sync_copy(acc_sh.at[pl.ds(s * RPT, RPT)],
                        out_hbm.at[c, pl.ds(s * RPT, RPT)])

    return k(x, idx_all, zeros_init)


R = 1000


def _tc1_body(x_ref, p_ref, ws_ref, wn_ref, b_ref, h_ref, inv_ref):
    acc = p_ref[0] + p_ref[1]
    deg = acc[:, IN_DIM:IN_DIM + 1]
    inv = 1.0 / jnp.maximum(deg, 1.0)
    agg = acc[:, :IN_DIM] * inv
    h = jnp.dot(x_ref[...], ws_ref[...], preferred_element_type=jnp.float32)
    h = h + jnp.dot(agg, wn_ref[...], preferred_element_type=jnp.float32)
    h = jnp.maximum(h + b_ref[...], 0.0)
    h_ref[0] = h[:, :D2]
    h_ref[1] = h[:, D2:]
    inv_ref[...] = jnp.broadcast_to(inv, (R, D2))


def _tc_layer1(x, part, Ws, Wn, b):
    return pl.pallas_call(
        _tc1_body,
        grid=(N // R,),
        in_specs=[
            pl.BlockSpec((R, IN_DIM), lambda i: (i, 0)),
            pl.BlockSpec((NC, R, D1), lambda i: (0, i, 0)),
            pl.BlockSpec((IN_DIM, H_DIM), lambda i: (0, 0)),
            pl.BlockSpec((IN_DIM, H_DIM), lambda i: (0, 0)),
            pl.BlockSpec((1, H_DIM), lambda i: (0, 0)),
        ],
        out_specs=[
            pl.BlockSpec((NC, R, D2), lambda i: (0, i, 0)),
            pl.BlockSpec((R, D2), lambda i: (i, 0)),
        ],
        out_shape=[
            jax.ShapeDtypeStruct((NC, N, D2), jnp.float32),
            jax.ShapeDtypeStruct((N, D2), jnp.float32),
        ],
    )(x, part, Ws, Wn, b)


def _tc2_body(h_ref, q_ref, inv_ref, ws_ref, wn_ref, b_ref, o_ref):
    inv = inv_ref[...]
    out = jnp.dot(h_ref[0], ws_ref[:D2], preferred_element_type=jnp.float32)
    out = out + jnp.dot(h_ref[1], ws_ref[D2:],
                        preferred_element_type=jnp.float32)
    out = out + jnp.dot(q_ref[0] * inv, wn_ref[:D2],
                        preferred_element_type=jnp.float32)
    out = out + jnp.dot(q_ref[1] * inv, wn_ref[D2:],
                        preferred_element_type=jnp.float32)
    o_ref[...] = out + b_ref[...]


def _tc_layer2(h, q, inv, Ws, Wn, b):
    return pl.pallas_call(
        _tc2_body,
        grid=(N // R,),
        in_specs=[
            pl.BlockSpec((NC, R, D2), lambda i: (0, i, 0)),
            pl.BlockSpec((NC, R, D2), lambda i: (0, i, 0)),
            pl.BlockSpec((R, D2), lambda i: (i, 0)),
            pl.BlockSpec((H_DIM, OUT_DIM), lambda i: (0, 0)),
            pl.BlockSpec((H_DIM, OUT_DIM), lambda i: (0, 0)),
            pl.BlockSpec((1, OUT_DIM), lambda i: (0, 0)),
        ],
        out_specs=pl.BlockSpec((R, OUT_DIM), lambda i: (i, 0)),
        out_shape=jax.ShapeDtypeStruct((N, OUT_DIM), jnp.float32),
    )(h, q, inv, Ws, Wn, b)


def kernel(feats, edge_index, W_self1, W_neigh1, b1, W_self2, W_neigh2, b2):
    idx_all = jnp.concatenate(
        [edge_index.reshape(2, E // C, C).transpose(1, 0, 2),
         jnp.zeros((2, 2, C), jnp.int32)], axis=0)
    pad = jnp.concatenate(
        [jnp.ones((N, 1), jnp.float32), jnp.zeros((N, 15), jnp.float32)],
        axis=1)
    x_aug = jnp.concatenate([feats, pad], axis=1)
    part1 = _sc_segsum(x_aug, idx_all, jnp.zeros((NP, D1), jnp.float32),
                       D1, split_edges_by_core=True)
    h, inv = _tc_layer1(feats, part1, W_self1, W_neigh1, b1.reshape(1, -1))
    part2 = _sc_segsum(h, idx_all, jnp.zeros((NP, D2), jnp.float32),
                       D2, split_edges_by_core=False)
    return _tc_layer2(h, part2, inv, W_self2, W_neigh2, b2.reshape(1, -1))


# speedup vs baseline: 12.1470x; 1.2832x over previous
import functools

import jax
import jax.numpy as jnp
from jax import lax
from jax.experimental import pallas as pl
from jax.experimental.pallas import tpu as pltpu
from jax.experimental.pallas import tpu_sc as plsc

N = 10000
NP = 10240
E = 320000
EP = 327680
IN_DIM = 128
H_DIM = 256
OUT_DIM = 256

NC = 2
NS = 16
D = 128
DD = 8

CR = 128
C2 = 64
NBUF = 5
IB = 10
NROW = EP // CR
RPT = NP // NS


def _sc_segsum(x, idx3, z128, z8, ones8, split_edges_by_core):
    with_deg = split_edges_by_core
    ept = EP // (NC * NS) if split_edges_by_core else EP // NS
    T = ept // C2
    outer = T // IB
    mesh = plsc.VectorSubcoreMesh(core_axis_name="c", subcore_axis_name="s")

    out_type = [jax.ShapeDtypeStruct((NC, NP, D), jnp.float32)]
    scratch = [
        pltpu.VMEM((IB, 1, C2), jnp.int32),
        pltpu.VMEM((IB, 1, C2), jnp.int32),
        pltpu.VMEM((NBUF, C2, D), jnp.float32),
        pltpu.VMEM_SHARED((NP, D), jnp.float32),
        pltpu.SemaphoreType.DMA((IB,)),
        pltpu.SemaphoreType.DMA((NBUF,)),
        pltpu.SemaphoreType.DMA((NBUF,)),
    ]
    if with_deg:
        out_type.append(jax.ShapeDtypeStruct((NC, NP, DD), jnp.float32))
        scratch += [
            pltpu.VMEM((C2, DD), jnp.float32),
            pltpu.VMEM_SHARED((NP, DD), jnp.float32),
            pltpu.SemaphoreType.DMA((NBUF,)),
        ]

    @functools.partial(
        pl.kernel,
        out_type=out_type,
        mesh=mesh,
        compiler_params=pltpu.CompilerParams(use_tc_tiling_on_sc=False),
        scratch_types=scratch,
    )
    def k(x_hbm, idx_hbm, z_hbm, z8_hbm, ones_hbm, *refs):
        if with_deg:
            (out_hbm, deg_hbm, srcv, dstv, rowsv, acc_sh,
             isem, gsem, ssem, onesv, acc8_sh, dsem) = refs
        else:
            (out_hbm, srcv, dstv, rowsv, acc_sh, isem, gsem, ssem) = refs
        c = lax.axis_index("c")
        s = lax.axis_index("s")
        wid = c * NS + s
        cbase = (wid if split_edges_by_core else s) * (ept // CR)
        table = x_hbm if split_edges_by_core else x_hbm.at[c]
        pltpu.sync_copy(z_hbm.at[pl.ds(s * RPT, RPT)],
                        acc_sh.at[pl.ds(s * RPT, RPT)])
        if with_deg:
            pltpu.sync_copy(z8_hbm.at[pl.ds(s * RPT, RPT)],
                            acc8_sh.at[pl.ds(s * RPT, RPT)])
            pltpu.sync_copy(ones_hbm, onesv)
        plsc.subcore_barrier()

        def idx_row(t_row):
            return jnp.minimum(cbase + t_row, NROW - 1)

        def idx_start(g, mm):
            rr = idx_row(g * (IB // 2) + (mm // 2))
            half = (mm % 2) * C2
            bi = mm % IB
            pltpu.async_copy(idx_hbm.at[rr, 0, pl.ds(half, C2)],
                             srcv.at[bi, 0], isem.at[bi])
            pltpu.async_copy(idx_hbm.at[rr, 1, pl.ds(half, C2)],
                             dstv.at[bi, 0], isem.at[bi])

        def idx_wait(g, mm):
            rr = idx_row(g * (IB // 2) + (mm // 2))
            half = (mm % 2) * C2
            bi = mm % IB
            pltpu.make_async_copy(idx_hbm.at[rr, 0, pl.ds(half, C2)],
                                  srcv.at[bi, 0], isem.at[bi]).wait()
            pltpu.make_async_copy(idx_hbm.at[rr, 1, pl.ds(half, C2)],
                                  dstv.at[bi, 0], isem.at[bi]).wait()

        def gather_start(b, bi):
            pltpu.async_copy(table.at[srcv.at[bi, 0]], rowsv.at[b],
                             gsem.at[b])

        def gather_wait(b, bi):
            pltpu.make_async_copy(table.at[srcv.at[bi, 0]], rowsv.at[b],
                                  gsem.at[b]).wait()

        def scatter_start(b, bi):
            pltpu.async_copy(rowsv.at[b], acc_sh.at[dstv.at[bi, 0]],
                             ssem.at[b], add=True)
            if with_deg:
                pltpu.async_copy(onesv, acc8_sh.at[dstv.at[bi, 0]],
                                 dsem.at[b], add=True)

        def scatter_wait(b, bi):
            pltpu.make_async_copy(rowsv.at[b], acc_sh.at[dstv.at[bi, 0]],
                                  ssem.at[b]).wait()
            if with_deg:
                pltpu.make_async_copy(onesv, acc8_sh.at[dstv.at[bi, 0]],
                                      dsem.at[b]).wait()

        idx_start(0, 0)
        idx_start(0, 1)
        for m in range(IB):
            if m + 2 < IB:
                idx_start(0, m + 2)
            else:
                idx_start(1, m + 2 - IB)
            if m >= NBUF:
                scatter_wait(m % NBUF, (m - NBUF) % IB)
            idx_wait(0, m)
            gather_start(m % NBUF, m)
            if m >= 2:
                gather_wait((m - 2) % NBUF, m - 2)
                scatter_start((m - 2) % NBUF, m - 2)

        def body(g, carry):
            for mm in range(IB):
                if mm + 2 < IB:
                    idx_start(g, mm + 2)
                else:
                    idx_start(g + 1, mm + 2 - IB)
                scatter_wait(mm % NBUF, (mm - NBUF) % IB)
                idx_wait(g, mm)
                gather_start(mm % NBUF, mm)
                b2 = (mm - 2) % NBUF
                bi2 = (mm - 2) % IB
                gather_wait(b2, bi2)
                scatter_start(b2, bi2)
            return carry

        lax.fori_loop(1, outer, body, 0)

        for m in (IB - 2, IB - 1):
            gather_wait(m % NBUF, m)
            scatter_start(m % NBUF, m)
        for b in range(NBUF):
            scatter_wait(b, b)
        idx_wait(outer, 0)
        idx_wait(outer, 1)
        plsc.subcore_barrier()
        pltpu.sync_copy(acc_sh.at[pl.ds(s * RPT, RPT)],
                        out_hbm.at[c, pl.ds(s * RPT, RPT)])
        if with_deg:
            pltpu.sync_copy(acc8_sh.at[pl.ds(s * RPT, RPT)],
                            deg_hbm.at[c, pl.ds(s * RPT, RPT)])

    return k(x, idx3, z128, z8, ones8)


R = 1000


def _tc1_body(x_ref, p_ref, d_ref, ws_ref, wn_ref, b_ref, h_ref, inv_ref):
    deg = d_ref[0][:, 0:1] + d_ref[1][:, 0:1]
    inv = 1.0 / jnp.maximum(deg, 1.0)
    agg = (p_ref[0] + p_ref[1]) * inv
    h = jnp.dot(x_ref[...], ws_ref[...], preferred_element_type=jnp.float32)
    h = h + jnp.dot(agg, wn_ref[...], preferred_element_type=jnp.float32)
    h = jnp.maximum(h + b_ref[...], 0.0)
    h_ref[0] = h[:, :D]
    h_ref[1] = h[:, D:]
    inv_ref[...] = jnp.broadcast_to(inv, (R, D))


def _tc_layer1(x, part, deg8, Ws, Wn, b):
    return pl.pallas_call(
        _tc1_body,
        grid=(N // R,),
        in_specs=[
            pl.BlockSpec((R, IN_DIM), lambda i: (i, 0)),
            pl.BlockSpec((NC, R, D), lambda i: (0, i, 0)),
            pl.BlockSpec((NC, R, DD), lambda i: (0, i, 0)),
            pl.BlockSpec((IN_DIM, H_DIM), lambda i: (0, 0)),
            pl.BlockSpec((IN_DIM, H_DIM), lambda i: (0, 0)),
            pl.BlockSpec((1, H_DIM), lambda i: (0, 0)),
        ],
        out_specs=[
            pl.BlockSpec((NC, R, D), lambda i: (0, i, 0)),
            pl.BlockSpec((R, D), lambda i: (i, 0)),
        ],
        out_shape=[
            jax.ShapeDtypeStruct((NC, N, D), jnp.float32),
            jax.ShapeDtypeStruct((N, D), jnp.float32),
        ],
    )(x, part, deg8, Ws, Wn, b)


def _tc2_body(h_ref, q_ref, inv_ref, ws_ref, wn_ref, b_ref, o_ref):
    inv = inv_ref[...]
    out = jnp.dot(h_ref[0], ws_ref[:D], preferred_element_type=jnp.float32)
    out = out + jnp.dot(h_ref[1], ws_ref[D:],
                        preferred_element_type=jnp.float32)
    out = out + jnp.dot(q_ref[0] * inv, wn_ref[:D],
                        preferred_element_type=jnp.float32)
    out = out + jnp.dot(q_ref[1] * inv, wn_ref[D:],
                        preferred_element_type=jnp.float32)
    o_ref[...] = out + b_ref[...]


def _tc_layer2(h, q, inv, Ws, Wn, b):
    return pl.pallas_call(
        _tc2_body,
        grid=(N // R,),
        in_specs=[
            pl.BlockSpec((NC, R, D), lambda i: (0, i, 0)),
            pl.BlockSpec((NC, R, D), lambda i: (0, i, 0)),
            pl.BlockSpec((R, D), lambda i: (i, 0)),
            pl.BlockSpec((H_DIM, OUT_DIM), lambda i: (0, 0)),
            pl.BlockSpec((H_DIM, OUT_DIM), lambda i: (0, 0)),
            pl.BlockSpec((1, OUT_DIM), lambda i: (0, 0)),
        ],
        out_specs=pl.BlockSpec((R, OUT_DIM), lambda i: (i, 0)),
        out_shape=jax.ShapeDtypeStruct((N, OUT_DIM), jnp.float32),
    )(h, q, inv, Ws, Wn, b)


def kernel(feats, edge_index, W_self1, W_neigh1, b1, W_self2, W_neigh2, b2):
    npad = EP - E
    pad_src = (jnp.arange(npad, dtype=jnp.int32) * 131) % N
    pad_dst = N + (jnp.arange(npad, dtype=jnp.int32) % (NP - N))
    eidx = jnp.concatenate(
        [edge_index, jnp.stack([pad_src, pad_dst])], axis=1)
    idx3 = eidx.reshape(2, NROW, CR).transpose(1, 0, 2)
    z128 = jnp.zeros((NP, D), jnp.float32)
    z8 = jnp.zeros((NP, DD), jnp.float32)
    ones8 = jnp.ones((C2, DD), jnp.float32)
    part1, deg8 = _sc_segsum(feats, idx3, z128, z8, ones8,
                             split_edges_by_core=True)
    h, inv = _tc_layer1(feats, part1, deg8, W_self1, W_neigh1,
                        b1.reshape(1, -1))
    part2, = _sc_segsum(h, idx3, z128, z8, ones8,
                        split_edges_by_core=False)
    return _tc_layer2(h, part2, inv, W_self2, W_neigh2, b2.reshape(1, -1))


# speedup vs baseline: 12.2358x; 1.0073x over previous
import functools

import jax
import jax.numpy as jnp
import numpy as np
from jax import lax
from jax.experimental import pallas as pl
from jax.experimental.pallas import tpu as pltpu
from jax.experimental.pallas import tpu_sc as plsc

N = 10000
NP = 10240
E = 320000
EP = 327680
IN_DIM = 128
H_DIM = 256
OUT_DIM = 256

NC = 2
NS = 16
D = 128
DD = 8

CR = 128
C2 = 64
NBUF = 5
IB = 10
NROW = EP // CR
RPT = NP // NS


def _sc_segsum(x, idx3, z128, z8, ones8, split_edges_by_core):
    with_deg = split_edges_by_core
    ept = EP // (NC * NS) if split_edges_by_core else EP // NS
    T = ept // C2
    outer = T // IB
    mesh = plsc.VectorSubcoreMesh(core_axis_name="c", subcore_axis_name="s")

    out_type = [jax.ShapeDtypeStruct((NC, NP, D), jnp.float32)]
    scratch = [
        pltpu.VMEM((IB, 1, C2), jnp.int32),
        pltpu.VMEM((IB, 1, C2), jnp.int32),
        pltpu.VMEM((NBUF, C2, D), jnp.float32),
        pltpu.VMEM_SHARED((NP, D), jnp.float32),
        pltpu.SemaphoreType.DMA((IB,)),
        pltpu.SemaphoreType.DMA((NBUF,)),
        pltpu.SemaphoreType.DMA((NBUF,)),
    ]
    if with_deg:
        out_type.append(jax.ShapeDtypeStruct((NC, NP, DD), jnp.float32))
        scratch += [
            pltpu.VMEM((C2, DD), jnp.float32),
            pltpu.VMEM_SHARED((NP, DD), jnp.float32),
            pltpu.SemaphoreType.DMA((NBUF,)),
        ]

    @functools.partial(
        pl.kernel,
        out_type=out_type,
        mesh=mesh,
        compiler_params=pltpu.CompilerParams(use_tc_tiling_on_sc=False),
        scratch_types=scratch,
    )
    def k(x_hbm, idx_hbm, z_hbm, z8_hbm, ones_hbm, *refs):
        if with_deg:
            (out_hbm, deg_hbm, srcv, dstv, rowsv, acc_sh,
             isem, gsem, ssem, onesv, acc8_sh, dsem) = refs
        else:
            (out_hbm, srcv, dstv, rowsv, acc_sh, isem, gsem, ssem) = refs
        c = lax.axis_index("c")
        s = lax.axis_index("s")
        wid = c * NS + s
        cbase = (wid if split_edges_by_core else s) * (ept // CR)
        table = x_hbm if split_edges_by_core else x_hbm.at[c]
        pltpu.sync_copy(z_hbm.at[pl.ds(s * RPT, RPT)],
                        acc_sh.at[pl.ds(s * RPT, RPT)])
        if with_deg:
            pltpu.sync_copy(z8_hbm.at[pl.ds(s * RPT, RPT)],
                            acc8_sh.at[pl.ds(s * RPT, RPT)])
            pltpu.sync_copy(ones_hbm, onesv)
        plsc.subcore_barrier()

        def idx_row(t_row):
            return jnp.minimum(cbase + t_row, NROW - 1)

        def idx_start(g, mm):
            rr = idx_row(g * (IB // 2) + (mm // 2))
            half = (mm % 2) * C2
            bi = mm % IB
            pltpu.async_copy(idx_hbm.at[rr, 0, pl.ds(half, C2)],
                             srcv.at[bi, 0], isem.at[bi])
            pltpu.async_copy(idx_hbm.at[rr, 1, pl.ds(half, C2)],
                             dstv.at[bi, 0], isem.at[bi])

        def idx_wait(g, mm):
            rr = idx_row(g * (IB // 2) + (mm // 2))
            half = (mm % 2) * C2
            bi = mm % IB
            pltpu.make_async_copy(idx_hbm.at[rr, 0, pl.ds(half, C2)],
                                  srcv.at[bi, 0], isem.at[bi]).wait()
            pltpu.make_async_copy(idx_hbm.at[rr, 1, pl.ds(half, C2)],
                                  dstv.at[bi, 0], isem.at[bi]).wait()

        def gather_start(b, bi):
            pltpu.async_copy(table.at[srcv.at[bi, 0]], rowsv.at[b],
                             gsem.at[b])

        def gather_wait(b, bi):
            pltpu.make_async_copy(table.at[srcv.at[bi, 0]], rowsv.at[b],
                                  gsem.at[b]).wait()

        def scatter_start(b, bi):
            pltpu.async_copy(rowsv.at[b], acc_sh.at[dstv.at[bi, 0]],
                             ssem.at[b], add=True)
            if with_deg:
                pltpu.async_copy(onesv, acc8_sh.at[dstv.at[bi, 0]],
                                 dsem.at[b], add=True)

        def scatter_wait(b, bi):
            pltpu.make_async_copy(rowsv.at[b], acc_sh.at[dstv.at[bi, 0]],
                                  ssem.at[b]).wait()
            if with_deg:
                pltpu.make_async_copy(onesv, acc8_sh.at[dstv.at[bi, 0]],
                                      dsem.at[b]).wait()

        idx_start(0, 0)
        idx_start(0, 1)
        for m in range(IB):
            if m + 2 < IB:
                idx_start(0, m + 2)
            else:
                idx_start(1, m + 2 - IB)
            if m >= NBUF:
                scatter_wait(m % NBUF, (m - NBUF) % IB)
            idx_wait(0, m)
            gather_start(m % NBUF, m)
            if m >= 2:
                gather_wait((m - 2) % NBUF, m - 2)
                scatter_start((m - 2) % NBUF, m - 2)

        def body(g, carry):
            for mm in range(IB):
                if mm + 2 < IB:
                    idx_start(g, mm + 2)
                else:
                    idx_start(g + 1, mm + 2 - IB)
                scatter_wait(mm % NBUF, (mm - NBUF) % IB)
                idx_wait(g, mm)
                gather_start(mm % NBUF, mm)
                b2 = (mm - 2) % NBUF
                bi2 = (mm - 2) % IB
                gather_wait(b2, bi2)
                scatter_start(b2, bi2)
            return carry

        lax.fori_loop(1, outer, body, 0)

        for m in (IB - 2, IB - 1):
            gather_wait(m % NBUF, m)
            scatter_start(m % NBUF, m)
        for b in range(NBUF):
            scatter_wait(b, b)
        idx_wait(outer, 0)
        idx_wait(outer, 1)
        plsc.subcore_barrier()
        pltpu.sync_copy(acc_sh.at[pl.ds(s * RPT, RPT)],
                        out_hbm.at[c, pl.ds(s * RPT, RPT)])
        if with_deg:
            pltpu.sync_copy(acc8_sh.at[pl.ds(s * RPT, RPT)],
                            deg_hbm.at[c, pl.ds(s * RPT, RPT)])

    return k(x, idx3, z128, z8, ones8)


R1B = 2000
R2B = 2000


def _tcs_body(x_ref, ws_ref, b_ref, o_ref):
    o_ref[...] = jnp.dot(x_ref[...], ws_ref[...],
                         preferred_element_type=jnp.float32) + b_ref[...]


def _tc_self(x, Ws, b, din, dout):
    return pl.pallas_call(
        _tcs_body,
        grid=(N // R2B,),
        in_specs=[
            pl.BlockSpec((R2B, din), lambda i: (i, 0)),
            pl.BlockSpec((din, dout), lambda i: (0, 0)),
            pl.BlockSpec((1, dout), lambda i: (0, 0)),
        ],
        out_specs=pl.BlockSpec((R2B, dout), lambda i: (i, 0)),
        out_shape=jax.ShapeDtypeStruct((N, dout), jnp.float32),
    )(x, Ws, b)


def _tcs2_body(h_ref, ws_ref, b_ref, o_ref):
    out = jnp.dot(h_ref[0], ws_ref[:D], preferred_element_type=jnp.float32)
    out = out + jnp.dot(h_ref[1], ws_ref[D:],
                        preferred_element_type=jnp.float32)
    o_ref[...] = out + b_ref[...]


def _tc_self2(h, Ws, b):
    return pl.pallas_call(
        _tcs2_body,
        grid=(N // R2B,),
        in_specs=[
            pl.BlockSpec((NC, R2B, D), lambda i: (0, i, 0)),
            pl.BlockSpec((H_DIM, OUT_DIM), lambda i: (0, 0)),
            pl.BlockSpec((1, OUT_DIM), lambda i: (0, 0)),
        ],
        out_specs=pl.BlockSpec((R2B, OUT_DIM), lambda i: (i, 0)),
        out_shape=jax.ShapeDtypeStruct((N, OUT_DIM), jnp.float32),
    )(h, Ws, b)


def _tc1_body(s_ref, p_ref, d_ref, wn_ref, h_ref, inv_ref):
    deg = d_ref[0][:, 0:1] + d_ref[1][:, 0:1]
    inv = 1.0 / jnp.maximum(deg, 1.0)
    agg = (p_ref[0] + p_ref[1]) * inv
    h = s_ref[...] + jnp.dot(agg, wn_ref[...],
                             preferred_element_type=jnp.float32)
    h = jnp.maximum(h, 0.0)
    h_ref[0] = h[:, :D]
    h_ref[1] = h[:, D:]
    inv_ref[...] = jnp.broadcast_to(inv, (R1B, D))


def _tc_layer1(s1, part, degv, Wn):
    return pl.pallas_call(
        _tc1_body,
        grid=(N // R1B,),
        in_specs=[
            pl.BlockSpec((R1B, H_DIM), lambda i: (i, 0)),
            pl.BlockSpec((NC, R1B, D), lambda i: (0, i, 0)),
            pl.BlockSpec((NC, R1B, DD), lambda i: (0, i, 0)),
            pl.BlockSpec((IN_DIM, H_DIM), lambda i: (0, 0)),
        ],
        out_specs=[
            pl.BlockSpec((NC, R1B, D), lambda i: (0, i, 0)),
            pl.BlockSpec((R1B, D), lambda i: (i, 0)),
        ],
        out_shape=[
            jax.ShapeDtypeStruct((NC, N, D), jnp.float32),
            jax.ShapeDtypeStruct((N, D), jnp.float32),
        ],
    )(s1, part, degv, Wn)


def _tc2_body(s_ref, q_ref, inv_ref, wn_ref, o_ref):
    inv = inv_ref[...]
    out = s_ref[...] + jnp.dot(q_ref[0] * inv, wn_ref[:D],
                               preferred_element_type=jnp.float32)
    out = out + jnp.dot(q_ref[1] * inv, wn_ref[D:],
                        preferred_element_type=jnp.float32)
    o_ref[...] = out


def _tc_layer2(s2, q, inv, Wn):
    return pl.pallas_call(
        _tc2_body,
        grid=(N // R1B,),
        in_specs=[
            pl.BlockSpec((R1B, OUT_DIM), lambda i: (i, 0)),
            pl.BlockSpec((NC, R1B, D), lambda i: (0, i, 0)),
            pl.BlockSpec((R1B, D), lambda i: (i, 0)),
            pl.BlockSpec((H_DIM, OUT_DIM), lambda i: (0, 0)),
        ],
        out_specs=pl.BlockSpec((R1B, OUT_DIM), lambda i: (i, 0)),
        out_shape=jax.ShapeDtypeStruct((N, OUT_DIM), jnp.float32),
    )(s2, q, inv, Wn)


_PAD_NP = EP - E
_PAD_EDGES = np.stack([
    (np.arange(_PAD_NP) * 131) % N,
    N + (np.arange(_PAD_NP) % (NP - N)),
]).astype(np.int32)


def kernel(feats, edge_index, W_self1, W_neigh1, b1, W_self2, W_neigh2, b2):
    eidx = jnp.concatenate([edge_index, jnp.asarray(_PAD_EDGES)], axis=1)
    idx3 = eidx.reshape(2, NROW, CR).transpose(1, 0, 2)
    z128 = jnp.zeros((NP, D), jnp.float32)
    z8 = jnp.zeros((NP, DD), jnp.float32)
    ones8 = jnp.ones((C2, DD), jnp.float32)
    part1, deg8 = _sc_segsum(feats, idx3, z128, z8, ones8,
                             split_edges_by_core=True)
    s1 = _tc_self(feats, W_self1, b1.reshape(1, -1), IN_DIM, H_DIM)
    h, inv = _tc_layer1(s1, part1, deg8, W_neigh1)
    part2, = _sc_segsum(h, idx3, z128, z8, ones8,
                        split_edges_by_core=False)
    s2 = _tc_self2(h, W_self2, b2.reshape(1, -1))
    return _tc_layer2(s2, part2, inv, W_neigh2)


# speedup vs baseline: 12.5052x; 1.0220x over previous
import functools

import jax
import jax.numpy as jnp
import numpy as np
from jax import lax
from jax.experimental import pallas as pl
from jax.experimental.pallas import tpu as pltpu
from jax.experimental.pallas import tpu_sc as plsc

N = 10000
NP = 10240
E = 320000
IN_DIM = 128
H_DIM = 256
OUT_DIM = 256

NC = 2
NS = 16
D = 128
DD = 8

CR = 128
C2 = 64
NBUF = 5
IB = 10
NROW = E // CR
RPT = NP // NS


def _sc_segsum(x, idx3, z128, z8, ones8, split_edges_by_core):
    with_deg = split_edges_by_core
    nt = NC * NS if split_edges_by_core else NS
    rows_pt = -(-NROW // (nt * (IB // 2))) * (IB // 2)
    mesh = plsc.VectorSubcoreMesh(core_axis_name="c", subcore_axis_name="s")

    out_type = [jax.ShapeDtypeStruct((NC, NP, D), jnp.float32)]
    scratch = [
        pltpu.VMEM((IB, 1, C2), jnp.int32),
        pltpu.VMEM((IB, 1, C2), jnp.int32),
        pltpu.VMEM((NBUF, C2, D), jnp.float32),
        pltpu.VMEM_SHARED((NP, D), jnp.float32),
        pltpu.SemaphoreType.DMA((IB,)),
        pltpu.SemaphoreType.DMA((NBUF,)),
        pltpu.SemaphoreType.DMA((NBUF,)),
    ]
    if with_deg:
        out_type.append(jax.ShapeDtypeStruct((NC, NP, DD), jnp.float32))
        scratch += [
            pltpu.VMEM((C2, DD), jnp.float32),
            pltpu.VMEM_SHARED((NP, DD), jnp.float32),
            pltpu.SemaphoreType.DMA((NBUF,)),
        ]

    @functools.partial(
        pl.kernel,
        out_type=out_type,
        mesh=mesh,
        compiler_params=pltpu.CompilerParams(use_tc_tiling_on_sc=False),
        scratch_types=scratch,
    )
    def k(x_hbm, idx_hbm, z_hbm, z8_hbm, ones_hbm, *refs):
        if with_deg:
            (out_hbm, deg_hbm, srcv, dstv, rowsv, acc_sh,
             isem, gsem, ssem, onesv, acc8_sh, dsem) = refs
        else:
            (out_hbm, srcv, dstv, rowsv, acc_sh, isem, gsem, ssem) = refs
        c = lax.axis_index("c")
        s = lax.axis_index("s")
        wid = c * NS + s
        cbase = (wid if split_edges_by_core else s) * rows_pt
        rows_real = jnp.clip(NROW - cbase, 0, rows_pt)
        outer = 2 * rows_real // IB
        table = x_hbm if split_edges_by_core else x_hbm.at[c]
        pltpu.sync_copy(z_hbm.at[pl.ds(s * RPT, RPT)],
                        acc_sh.at[pl.ds(s * RPT, RPT)])
        if with_deg:
            pltpu.sync_copy(z8_hbm.at[pl.ds(s * RPT, RPT)],
                            acc8_sh.at[pl.ds(s * RPT, RPT)])
            pltpu.sync_copy(ones_hbm, onesv)
        plsc.subcore_barrier()

        def idx_row(t_row):
            return jnp.minimum(cbase + t_row, NROW - 1)

        def idx_start(g, mm):
            rr = idx_row(g * (IB // 2) + (mm // 2))
            half = (mm % 2) * C2
            bi = mm % IB
            pltpu.async_copy(idx_hbm.at[rr, 0, pl.ds(half, C2)],
                             srcv.at[bi, 0], isem.at[bi])
            pltpu.async_copy(idx_hbm.at[rr, 1, pl.ds(half, C2)],
                             dstv.at[bi, 0], isem.at[bi])

        def idx_wait(g, mm):
            rr = idx_row(g * (IB // 2) + (mm // 2))
            half = (mm % 2) * C2
            bi = mm % IB
            pltpu.make_async_copy(idx_hbm.at[rr, 0, pl.ds(half, C2)],
                                  srcv.at[bi, 0], isem.at[bi]).wait()
            pltpu.make_async_copy(idx_hbm.at[rr, 1, pl.ds(half, C2)],
                                  dstv.at[bi, 0], isem.at[bi]).wait()

        def gather_start(b, bi):
            pltpu.async_copy(table.at[srcv.at[bi, 0]], rowsv.at[b],
                             gsem.at[b])

        def gather_wait(b, bi):
            pltpu.make_async_copy(table.at[srcv.at[bi, 0]], rowsv.at[b],
                                  gsem.at[b]).wait()

        def scatter_start(b, bi):
            pltpu.async_copy(rowsv.at[b], acc_sh.at[dstv.at[bi, 0]],
                             ssem.at[b], add=True)
            if with_deg:
                pltpu.async_copy(onesv, acc8_sh.at[dstv.at[bi, 0]],
                                 dsem.at[b], add=True)

        def scatter_wait(b, bi):
            pltpu.make_async_copy(rowsv.at[b], acc_sh.at[dstv.at[bi, 0]],
                                  ssem.at[b]).wait()
            if with_deg:
                pltpu.make_async_copy(onesv, acc8_sh.at[dstv.at[bi, 0]],
                                      dsem.at[b]).wait()

        idx_start(0, 0)
        idx_start(0, 1)
        for m in range(IB):
            if m + 2 < IB:
                idx_start(0, m + 2)
            else:
                idx_start(1, m + 2 - IB)
            if m >= NBUF:
                scatter_wait(m % NBUF, (m - NBUF) % IB)
            idx_wait(0, m)
            gather_start(m % NBUF, m)
            if m >= 2:
                gather_wait((m - 2) % NBUF, m - 2)
                scatter_start((m - 2) % NBUF, m - 2)

        def body(g, carry):
            for mm in range(IB):
                if mm + 2 < IB:
                    idx_start(g, mm + 2)
                else:
                    idx_start(g + 1, mm + 2 - IB)
                scatter_wait(mm % NBUF, (mm - NBUF) % IB)
                idx_wait(g, mm)
                gather_start(mm % NBUF, mm)
                b2 = (mm - 2) % NBUF
                bi2 = (mm - 2) % IB
                gather_wait(b2, bi2)
                scatter_start(b2, bi2)
            return carry

        lax.fori_loop(1, outer, body, 0)

        for m in (IB - 2, IB - 1):
            gather_wait(m % NBUF, m)
            scatter_start(m % NBUF, m)
        for b in range(NBUF):
            scatter_wait(b, b)
        idx_wait(outer, 0)
        idx_wait(outer, 1)
        plsc.subcore_barrier()
        pltpu.sync_copy(acc_sh.at[pl.ds(s * RPT, RPT)],
                        out_hbm.at[c, pl.ds(s * RPT, RPT)])
        if with_deg:
            pltpu.sync_copy(acc8_sh.at[pl.ds(s * RPT, RPT)],
                            deg_hbm.at[c, pl.ds(s * RPT, RPT)])

    return k(x, idx3, z128, z8, ones8)


R1B = 2000
R2B = 2000


def _tcs_body(x_ref, ws_ref, b_ref, o_ref):
    o_ref[...] = jnp.dot(x_ref[...], ws_ref[...],
                         preferred_element_type=jnp.float32) + b_ref[...]


def _tc_self(x, Ws, b, din, dout):
    return pl.pallas_call(
        _tcs_body,
        grid=(N // R2B,),
        in_specs=[
            pl.BlockSpec((R2B, din), lambda i: (i, 0)),
            pl.BlockSpec((din, dout), lambda i: (0, 0)),
            pl.BlockSpec((1, dout), lambda i: (0, 0)),
        ],
        out_specs=pl.BlockSpec((R2B, dout), lambda i: (i, 0)),
        out_shape=jax.ShapeDtypeStruct((N, dout), jnp.float32),
    )(x, Ws, b)


def _tcs2_body(h_ref, ws_ref, b_ref, o_ref):
    out = jnp.dot(h_ref[0], ws_ref[:D], preferred_element_type=jnp.float32)
    out = out + jnp.dot(h_ref[1], ws_ref[D:],
                        preferred_element_type=jnp.float32)
    o_ref[...] = out + b_ref[...]


def _tc_self2(h, Ws, b):
    return pl.pallas_call(
        _tcs2_body,
        grid=(N // R2B,),
        in_specs=[
            pl.BlockSpec((NC, R2B, D), lambda i: (0, i, 0)),
            pl.BlockSpec((H_DIM, OUT_DIM), lambda i: (0, 0)),
            pl.BlockSpec((1, OUT_DIM), lambda i: (0, 0)),
        ],
        out_specs=pl.BlockSpec((R2B, OUT_DIM), lambda i: (i, 0)),
        out_shape=jax.ShapeDtypeStruct((N, OUT_DIM), jnp.float32),
    )(h, Ws, b)


def _tc1_body(s_ref, p_ref, d_ref, wn_ref, h_ref):
    deg = d_ref[0][:, 0:1] + d_ref[1][:, 0:1]
    inv = 1.0 / jnp.maximum(deg, 1.0)
    agg = (p_ref[0] + p_ref[1]) * inv
    h = s_ref[...] + jnp.dot(agg, wn_ref[...],
                             preferred_element_type=jnp.float32)
    h = jnp.maximum(h, 0.0)
    h_ref[0] = h[:, :D]
    h_ref[1] = h[:, D:]


def _tc_layer1(s1, part, degv, Wn):
    return pl.pallas_call(
        _tc1_body,
        grid=(N // R1B,),
        in_specs=[
            pl.BlockSpec((R1B, H_DIM), lambda i: (i, 0)),
            pl.BlockSpec((NC, R1B, D), lambda i: (0, i, 0)),
            pl.BlockSpec((NC, R1B, DD), lambda i: (0, i, 0)),
            pl.BlockSpec((IN_DIM, H_DIM), lambda i: (0, 0)),
        ],
        out_specs=pl.BlockSpec((NC, R1B, D), lambda i: (0, i, 0)),
        out_shape=jax.ShapeDtypeStruct((NC, N, D), jnp.float32),
    )(s1, part, degv, Wn)


def _tc2_body(s_ref, q_ref, d_ref, wn_ref, o_ref):
    deg = d_ref[0][:, 0:1] + d_ref[1][:, 0:1]
    inv = 1.0 / jnp.maximum(deg, 1.0)
    out = s_ref[...] + jnp.dot(q_ref[0] * inv, wn_ref[:D],
                               preferred_element_type=jnp.float32)
    out = out + jnp.dot(q_ref[1] * inv, wn_ref[D:],
                        preferred_element_type=jnp.float32)
    o_ref[...] = out


def _tc_layer2(s2, q, deg8, Wn):
    return pl.pallas_call(
        _tc2_body,
        grid=(N // R1B,),
        in_specs=[
            pl.BlockSpec((R1B, OUT_DIM), lambda i: (i, 0)),
            pl.BlockSpec((NC, R1B, D), lambda i: (0, i, 0)),
            pl.BlockSpec((NC, R1B, DD), lambda i: (0, i, 0)),
            pl.BlockSpec((H_DIM, OUT_DIM), lambda i: (0, 0)),
        ],
        out_specs=pl.BlockSpec((R1B, OUT_DIM), lambda i: (i, 0)),
        out_shape=jax.ShapeDtypeStruct((N, OUT_DIM), jnp.float32),
    )(s2, q, deg8, Wn)


def kernel(feats, edge_index, W_self1, W_neigh1, b1, W_self2, W_neigh2, b2):
    idx3 = edge_index.reshape(2, NROW, CR).transpose(1, 0, 2)
    z128 = jnp.zeros((NP, D), jnp.float32)
    z8 = jnp.zeros((NP, DD), jnp.float32)
    ones8 = jnp.ones((C2, DD), jnp.float32)
    part1, deg8 = _sc_segsum(feats, idx3, z128, z8, ones8,
                             split_edges_by_core=True)
    s1 = _tc_self(feats, W_self1, b1.reshape(1, -1), IN_DIM, H_DIM)
    h = _tc_layer1(s1, part1, deg8, W_neigh1)
    part2, = _sc_segsum(h, idx3, z128, z8, ones8,
                        split_edges_by_core=False)
    s2 = _tc_self2(h, W_self2, b2.reshape(1, -1))
    return _tc_layer2(s2, part2, deg8, W_neigh2)
